# baseline (speedup 1.0000x reference)
import functools

import jax
import jax.numpy as jnp
from jax import lax
from jax.experimental import pallas as pl
from jax.experimental.pallas import tpu as pltpu
from jax.experimental.pallas import tpu_sc as plsc

E = 160000
D = 256
L = 16
NW = 32
EPW = E // NW
C = 96
NCH = -(-EPW // C)
NROW = NCH + 1
CPAD = NCH * C


def _dot_body(h_hbm, src_hbm, dst_hbm, out_hbm, src_v, dst_v, u0, v0, u1, v1,
              out_v, sem0, sem1):
    wid = lax.axis_index("s") * 2 + lax.axis_index("c")
    base = wid * EPW
    pltpu.sync_copy(src_hbm.at[wid], src_v)
    pltpu.sync_copy(dst_hbm.at[wid], dst_v)

    def start(j, us, vs, sem):
        pltpu.async_copy(h_hbm.at[src_v.at[j]], us, sem)
        pltpu.async_copy(h_hbm.at[dst_v.at[j]], vs, sem)

    def wait(us, vs, sem):
        pltpu.make_async_copy(h_hbm.at[src_v.at[0]], us, sem).wait()
        pltpu.make_async_copy(h_hbm.at[src_v.at[0]], vs, sem).wait()

    lane = lax.iota(jnp.int32, L)

    def compute_chunk(u_v, v_v, j):
        for g in range(C // L):
            def edge_body(i, gvec, g=g):
                e = g * L + i
                p = [u_v[e, pl.ds(k * L, L)] * v_v[e, pl.ds(k * L, L)]
                     for k in range(D // L)]
                while len(p) > 1:
                    p = [p[a] + p[a + 1] for a in range(0, len(p), 2)]
                return jnp.where(lane == i, jnp.sum(p[0]), gvec)

            gvec = lax.fori_loop(0, L, edge_body,
                                 jnp.zeros((L,), jnp.float32))
            out_v[pl.ds(j * C + g * L, L)] = gvec

    start(0, u0, v0, sem0)
    start(1, u1, v1, sem1)

    def pair_body(i, _):
        j0 = 2 * i
        wait(u0, v0, sem0)
        compute_chunk(u0, v0, j0)
        start(j0 + 2, u0, v0, sem0)
        wait(u1, v1, sem1)
        compute_chunk(u1, v1, j0 + 1)
        start(j0 + 3, u1, v1, sem1)
        return 0

    lax.fori_loop(0, (NCH - 1) // 2, pair_body, 0)

    wait(u0, v0, sem0)
    compute_chunk(u0, v0, NCH - 1)
    wait(u1, v1, sem1)
    pltpu.sync_copy(out_v.at[pl.ds(0, EPW)], out_hbm.at[pl.ds(base, EPW)])


_dot_kernel = functools.partial(
    pl.kernel,
    out_type=jax.ShapeDtypeStruct((E,), jnp.float32),
    mesh=plsc.VectorSubcoreMesh(core_axis_name="c", subcore_axis_name="s"),
    compiler_params=pltpu.CompilerParams(needs_layout_passes=False),
    scratch_types=[
        pltpu.VMEM((NROW, C), jnp.int32),
        pltpu.VMEM((NROW, C), jnp.int32),
        pltpu.VMEM((C, D), jnp.float32),
        pltpu.VMEM((C, D), jnp.float32),
        pltpu.VMEM((C, D), jnp.float32),
        pltpu.VMEM((C, D), jnp.float32),
        pltpu.VMEM((CPAD,), jnp.float32),
        pltpu.SemaphoreType.DMA,
        pltpu.SemaphoreType.DMA,
    ],
)(_dot_body)


@jax.jit
def kernel(h, edge_index):
    pad = ((0, 0), (0, NROW * C - EPW))
    src = jnp.pad(edge_index[0].astype(jnp.int32).reshape(NW, EPW), pad)
    dst = jnp.pad(edge_index[1].astype(jnp.int32).reshape(NW, EPW), pad)
    return _dot_kernel(h, src.reshape(NW, NROW, C), dst.reshape(NW, NROW, C))


# speedup vs baseline: 1.0019x; 1.0019x over previous
import functools

import jax
import jax.numpy as jnp
from jax import lax
from jax.experimental import pallas as pl
from jax.experimental.pallas import tpu as pltpu
from jax.experimental.pallas import tpu_sc as plsc

E = 160000
D = 256
L = 16
NW = 32
EPW = E // NW
C = 96
NCH = -(-EPW // C)
NROW = NCH + 1
CPAD = NCH * C


def _dot_body(h_hbm, src_hbm, dst_hbm, out_hbm, src_v, dst_v, u0, v0, u1, v1,
              out_v, sem0, sem1):
    wid = lax.axis_index("s") * 2 + lax.axis_index("c")
    base = wid * EPW
    pltpu.sync_copy(src_hbm.at[wid], src_v)
    pltpu.sync_copy(dst_hbm.at[wid], dst_v)

    def start(j, us, vs, sem):
        pltpu.async_copy(h_hbm.at[src_v.at[j]], us, sem)
        pltpu.async_copy(h_hbm.at[dst_v.at[j]], vs, sem)

    def wait(us, vs, sem):
        pltpu.make_async_copy(h_hbm.at[src_v.at[0]], us, sem).wait()
        pltpu.make_async_copy(h_hbm.at[src_v.at[0]], vs, sem).wait()

    lane = lax.iota(jnp.int32, L)

    def compute_chunk(u_v, v_v, j):
        def group_body(g, _):
            def edge_body(i, gvec):
                e = g * L + i
                p = [u_v[e, pl.ds(k * L, L)] * v_v[e, pl.ds(k * L, L)]
                     for k in range(D // L)]
                while len(p) > 1:
                    p = [p[a] + p[a + 1] for a in range(0, len(p), 2)]
                return jnp.where(lane == i, jnp.sum(p[0]), gvec)

            gvec = lax.fori_loop(0, L, edge_body,
                                 jnp.zeros((L,), jnp.float32))
            out_v[pl.ds(j * C + g * L, L)] = gvec
            return 0

        lax.fori_loop(0, C // L, group_body, 0)

    start(0, u0, v0, sem0)
    start(1, u1, v1, sem1)

    def pair_body(i, _):
        j0 = 2 * i
        wait(u0, v0, sem0)
        compute_chunk(u0, v0, j0)
        start(j0 + 2, u0, v0, sem0)
        wait(u1, v1, sem1)
        compute_chunk(u1, v1, j0 + 1)
        start(j0 + 3, u1, v1, sem1)
        return 0

    lax.fori_loop(0, (NCH - 1) // 2, pair_body, 0)

    wait(u0, v0, sem0)
    compute_chunk(u0, v0, NCH - 1)
    wait(u1, v1, sem1)
    pltpu.sync_copy(out_v.at[pl.ds(0, EPW)], out_hbm.at[pl.ds(base, EPW)])


_dot_kernel = functools.partial(
    pl.kernel,
    out_type=jax.ShapeDtypeStruct((E,), jnp.float32),
    mesh=plsc.VectorSubcoreMesh(core_axis_name="c", subcore_axis_name="s"),
    compiler_params=pltpu.CompilerParams(needs_layout_passes=False),
    scratch_types=[
        pltpu.VMEM((NROW, C), jnp.int32),
        pltpu.VMEM((NROW, C), jnp.int32),
        pltpu.VMEM((C, D), jnp.float32),
        pltpu.VMEM((C, D), jnp.float32),
        pltpu.VMEM((C, D), jnp.float32),
        pltpu.VMEM((C, D), jnp.float32),
        pltpu.VMEM((CPAD,), jnp.float32),
        pltpu.SemaphoreType.DMA,
        pltpu.SemaphoreType.DMA,
    ],
)(_dot_body)


@jax.jit
def kernel(h, edge_index):
    pad = ((0, 0), (0, NROW * C - EPW))
    src = jnp.pad(edge_index[0].astype(jnp.int32).reshape(NW, EPW), pad)
    dst = jnp.pad(edge_index[1].astype(jnp.int32).reshape(NW, EPW), pad)
    return _dot_kernel(h, src.reshape(NW, NROW, C), dst.reshape(NW, NROW, C))


# speedup vs baseline: 1.5546x; 1.5517x over previous
import functools

import jax
import jax.numpy as jnp
from jax import lax
from jax.experimental import pallas as pl
from jax.experimental.pallas import tpu as pltpu
from jax.experimental.pallas import tpu_sc as plsc

E = 160000
D = 256
L = 16
NW = 32
EPW = E // NW
C = 48
NCH = -(-EPW // C)
NROW = NCH + 1
CPAD = NCH * C


def _dot_body(h_hbm, src_hbm, dst_hbm, out_hbm, src_v, dst_v, u0, v0, u1, v1,
              out_v, sem0, sem1):
    wid = lax.axis_index("s") * 2 + lax.axis_index("c")
    base = wid * EPW
    pltpu.sync_copy(src_hbm.at[wid], src_v)
    pltpu.sync_copy(dst_hbm.at[wid], dst_v)

    def start(j, us, vs, sem):
        pltpu.async_copy(h_hbm.at[src_v.at[j]], us, sem)
        pltpu.async_copy(h_hbm.at[dst_v.at[j]], vs, sem)

    def wait(us, vs, sem):
        pltpu.make_async_copy(h_hbm.at[src_v.at[0]], us, sem).wait()
        pltpu.make_async_copy(h_hbm.at[src_v.at[0]], vs, sem).wait()

    lane = lax.iota(jnp.int32, L)

    def compute_chunk(u_v, v_v, j):
        def group_body(g, _):
            def edge_body(i, gvec):
                e = g * L + i
                p = [u_v[e, pl.ds(k * L, L)] * v_v[e, pl.ds(k * L, L)]
                     for k in range(D // L)]
                while len(p) > 1:
                    p = [p[a] + p[a + 1] for a in range(0, len(p), 2)]
                return jnp.where(lane == i, jnp.sum(p[0]), gvec)

            gvec = lax.fori_loop(0, L, edge_body,
                                 jnp.zeros((L,), jnp.float32))
            out_v[pl.ds(j * C + g * L, L)] = gvec
            return 0

        lax.fori_loop(0, C // L, group_body, 0)

    start(0, u0, v0, sem0)
    start(1, u1, v1, sem1)

    def pair_body(i, _):
        j0 = 2 * i
        wait(u0, v0, sem0)
        compute_chunk(u0, v0, j0)
        start(j0 + 2, u0, v0, sem0)
        wait(u1, v1, sem1)
        compute_chunk(u1, v1, j0 + 1)
        start(j0 + 3, u1, v1, sem1)
        return 0

    lax.fori_loop(0, (NCH - 1) // 2, pair_body, 0)

    wait(u0, v0, sem0)
    compute_chunk(u0, v0, NCH - 1)
    wait(u1, v1, sem1)
    pltpu.sync_copy(out_v.at[pl.ds(0, EPW)], out_hbm.at[pl.ds(base, EPW)])


_dot_kernel = functools.partial(
    pl.kernel,
    out_type=jax.ShapeDtypeStruct((E,), jnp.float32),
    mesh=plsc.VectorSubcoreMesh(core_axis_name="c", subcore_axis_name="s"),
    compiler_params=pltpu.CompilerParams(needs_layout_passes=False),
    scratch_types=[
        pltpu.VMEM((NROW, C), jnp.int32),
        pltpu.VMEM((NROW, C), jnp.int32),
        pltpu.VMEM((C, D), jnp.float32),
        pltpu.VMEM((C, D), jnp.float32),
        pltpu.VMEM((C, D), jnp.float32),
        pltpu.VMEM((C, D), jnp.float32),
        pltpu.VMEM((CPAD,), jnp.float32),
        pltpu.SemaphoreType.DMA,
        pltpu.SemaphoreType.DMA,
    ],
)(_dot_body)


@jax.jit
def kernel(h, edge_index):
    pad = ((0, 0), (0, NROW * C - EPW))
    src = jnp.pad(edge_index[0].astype(jnp.int32).reshape(NW, EPW), pad)
    dst = jnp.pad(edge_index[1].astype(jnp.int32).reshape(NW, EPW), pad)
    return _dot_kernel(h, src.reshape(NW, NROW, C), dst.reshape(NW, NROW, C))


# speedup vs baseline: 1.9739x; 1.2697x over previous
import functools

import jax
import jax.numpy as jnp
from jax import lax
from jax.experimental import pallas as pl
from jax.experimental.pallas import tpu as pltpu
from jax.experimental.pallas import tpu_sc as plsc

E = 160000
D = 256
L = 16
NW = 32
EPW = E // NW
C = 16
NCH = -(-EPW // C)
NROW = NCH + 1
CPAD = NCH * C


def _dot_body(h_hbm, src_hbm, dst_hbm, out_hbm, src_v, dst_v, u0, v0, u1, v1,
              out_v, sem0, sem1):
    wid = lax.axis_index("s") * 2 + lax.axis_index("c")
    base = wid * EPW
    pltpu.sync_copy(src_hbm.at[wid], src_v)
    pltpu.sync_copy(dst_hbm.at[wid], dst_v)

    def start(j, us, vs, sem):
        pltpu.async_copy(h_hbm.at[src_v.at[j]], us, sem)
        pltpu.async_copy(h_hbm.at[dst_v.at[j]], vs, sem)

    def wait(us, vs, sem):
        pltpu.make_async_copy(h_hbm.at[src_v.at[0]], us, sem).wait()
        pltpu.make_async_copy(h_hbm.at[src_v.at[0]], vs, sem).wait()

    lane = lax.iota(jnp.int32, L)

    def compute_chunk(u_v, v_v, j):
        def group_body(g, _):
            def edge_body(i, gvec):
                e = g * L + i
                p = [u_v[e, pl.ds(k * L, L)] * v_v[e, pl.ds(k * L, L)]
                     for k in range(D // L)]
                while len(p) > 1:
                    p = [p[a] + p[a + 1] for a in range(0, len(p), 2)]
                return jnp.where(lane == i, jnp.sum(p[0]), gvec)

            gvec = lax.fori_loop(0, L, edge_body,
                                 jnp.zeros((L,), jnp.float32))
            out_v[pl.ds(j * C + g * L, L)] = gvec
            return 0

        lax.fori_loop(0, C // L, group_body, 0)

    start(0, u0, v0, sem0)
    start(1, u1, v1, sem1)

    def pair_body(i, _):
        j0 = 2 * i
        wait(u0, v0, sem0)
        compute_chunk(u0, v0, j0)
        start(j0 + 2, u0, v0, sem0)
        wait(u1, v1, sem1)
        compute_chunk(u1, v1, j0 + 1)
        start(j0 + 3, u1, v1, sem1)
        return 0

    lax.fori_loop(0, (NCH - 1) // 2, pair_body, 0)

    wait(u0, v0, sem0)
    compute_chunk(u0, v0, NCH - 1)
    wait(u1, v1, sem1)
    pltpu.sync_copy(out_v.at[pl.ds(0, EPW)], out_hbm.at[pl.ds(base, EPW)])


_dot_kernel = functools.partial(
    pl.kernel,
    out_type=jax.ShapeDtypeStruct((E,), jnp.float32),
    mesh=plsc.VectorSubcoreMesh(core_axis_name="c", subcore_axis_name="s"),
    compiler_params=pltpu.CompilerParams(needs_layout_passes=False),
    scratch_types=[
        pltpu.VMEM((NROW, C), jnp.int32),
        pltpu.VMEM((NROW, C), jnp.int32),
        pltpu.VMEM((C, D), jnp.float32),
        pltpu.VMEM((C, D), jnp.float32),
        pltpu.VMEM((C, D), jnp.float32),
        pltpu.VMEM((C, D), jnp.float32),
        pltpu.VMEM((CPAD,), jnp.float32),
        pltpu.SemaphoreType.DMA,
        pltpu.SemaphoreType.DMA,
    ],
)(_dot_body)


@jax.jit
def kernel(h, edge_index):
    pad = ((0, 0), (0, NROW * C - EPW))
    src = jnp.pad(edge_index[0].astype(jnp.int32).reshape(NW, EPW), pad)
    dst = jnp.pad(edge_index[1].astype(jnp.int32).reshape(NW, EPW), pad)
    return _dot_kernel(h, src.reshape(NW, NROW, C), dst.reshape(NW, NROW, C))
